# baseline (device time: 17225 ns/iter reference)
import jax
import jax.numpy as jnp
from jax import lax
from jax.experimental import pallas as pl
from jax.experimental.pallas import tpu as pltpu

M = 1024
HALF = 512
QROWS = 256
CROWS = 64
NCH = QROWS // CROWS
NPL = NCH + 2


def kernel(x):
    def body(
        x_ref,
        out_ref,
        in_all,
        send_y,
        recv_y,
        sum_buf,
        prx,
        prz,
        dsem,
        osem,
        ysend,
        yrecv,
        xsend,
        xrecv,
        zsend,
        zrecv,
    ):
        my_x = lax.axis_index("x")
        my_y = lax.axis_index("y")
        my_z = lax.axis_index("z")
        ypeer = (my_x, 1 - my_y, my_z)
        xnb = (1 - my_x, my_y, my_z)
        znb = (my_x, my_y, 1 - my_z)

        my_col = my_y * HALF
        peer_col = (1 - my_y) * HALF

        def qbase(qx, qz):
            return (2 * qx + qz) * QROWS

        base_me = qbase(my_x, my_z)
        base_x = qbase(1 - my_x, my_z)
        base_z = qbase(my_x, 1 - my_z)
        base_d = qbase(1 - my_x, 1 - my_z)

        barrier = pltpu.get_barrier_semaphore()
        for nbr in (ypeer, xnb, znb):
            pl.semaphore_signal(
                barrier, inc=1, device_id=nbr, device_id_type=pl.DeviceIdType.MESH
            )

        dmas = []
        for c in range(NCH):
            rows = pl.ds(base_me + c * CROWS, CROWS)
            dm = pltpu.make_async_copy(x_ref.at[0, rows, :], in_all.at[c], dsem.at[c])
            dm.start()
            dmas.append(dm)

        pl.semaphore_wait(barrier, 3)

        rdmas_y = []
        for c in range(NCH):
            dmas[c].wait()
            send_y[c] = in_all[c, :, pl.ds(peer_col, HALF)].astype(jnp.bfloat16)
            ry = pltpu.make_async_remote_copy(
                src_ref=send_y.at[c],
                dst_ref=recv_y.at[c],
                send_sem=ysend.at[c],
                recv_sem=yrecv.at[c],
                device_id=ypeer,
                device_id_type=pl.DeviceIdType.MESH,
            )
            ry.start()
            rdmas_y.append(ry)

        def remote(src, dst, ssem, rsem, dev):
            r = pltpu.make_async_remote_copy(
                src_ref=src, dst_ref=dst, send_sem=ssem, recv_sem=rsem,
                device_id=dev, device_id_type=pl.DeviceIdType.MESH,
            )
            r.start()
            return r

        copies = []

        def to_out(buf, row_start, sem_idx):
            cp = pltpu.make_async_copy(
                buf, out_ref.at[pl.ds(row_start, CROWS)], osem.at[sem_idx]
            )
            cp.start()
            copies.append(cp)

        sends = []
        for c in range(NCH):
            rdmas_y[c].wait()
            sum_buf[c] = (
                in_all[c, :, pl.ds(my_col, HALF)].astype(jnp.bfloat16) + recv_y[c]
            )
            sends.append(remote(sum_buf.at[c], prx.at[c], xsend.at[c], xrecv.at[c], xnb))
            sends.append(remote(sum_buf.at[c], prz.at[c], zsend.at[c], zrecv.at[c], znb))
            to_out(sum_buf.at[c], base_me + c * CROWS, c)

        def wait_in(buf_slot, rsem):
            r = pltpu.make_async_remote_copy(
                src_ref=sum_buf.at[0], dst_ref=buf_slot,
                send_sem=ysend.at[0], recv_sem=rsem,
                device_id=xnb, device_id_type=pl.DeviceIdType.MESH,
            )
            r.wait_recv()

        for i, c in enumerate((2, 3)):
            wait_in(prz.at[c], zrecv.at[c])
            sends.append(
                remote(prz.at[c], prx.at[NCH + i], xsend.at[NCH + i],
                       xrecv.at[NCH + i], xnb)
            )
            to_out(prz.at[c], base_z + c * CROWS, NCH + i)
        for i, c in enumerate((0, 1)):
            wait_in(prx.at[c], xrecv.at[c])
            sends.append(
                remote(prx.at[c], prz.at[NCH + i], zsend.at[NCH + i],
                       zrecv.at[NCH + i], znb)
            )
            to_out(prx.at[c], base_x + c * CROWS, NCH + 2 + i)

        for c in (2, 3):
            wait_in(prx.at[c], xrecv.at[c])
            to_out(prx.at[c], base_x + c * CROWS, NCH + 4 + (c - 2))
        for c in (0, 1):
            wait_in(prz.at[c], zrecv.at[c])
            to_out(prz.at[c], base_z + c * CROWS, NCH + 6 + c)

        for i in range(2):
            wait_in(prx.at[NCH + i], xrecv.at[NCH + i])
            to_out(prx.at[NCH + i], base_d + (2 + i) * CROWS, NCH + 8 + i)
            wait_in(prz.at[NCH + i], zrecv.at[NCH + i])
            to_out(prz.at[NCH + i], base_d + i * CROWS, NCH + 10 + i)

        for cp in copies:
            cp.wait()
        for s in sends:
            s.wait_send()

    return pl.pallas_call(
        body,
        out_shape=jax.ShapeDtypeStruct((M, HALF), jnp.bfloat16),
        in_specs=[pl.BlockSpec(memory_space=pl.ANY)],
        out_specs=pl.BlockSpec(memory_space=pl.ANY),
        scratch_shapes=[
            pltpu.VMEM((NCH, CROWS, 2 * HALF), jnp.float32),
            pltpu.VMEM((NCH, CROWS, HALF), jnp.bfloat16),
            pltpu.VMEM((NCH, CROWS, HALF), jnp.bfloat16),
            pltpu.VMEM((NCH, CROWS, HALF), jnp.bfloat16),
            pltpu.VMEM((NPL, CROWS, HALF), jnp.bfloat16),
            pltpu.VMEM((NPL, CROWS, HALF), jnp.bfloat16),
            pltpu.SemaphoreType.DMA((NCH,)),
            pltpu.SemaphoreType.DMA((16,)),
            pltpu.SemaphoreType.DMA((NCH,)),
            pltpu.SemaphoreType.DMA((NCH,)),
            pltpu.SemaphoreType.DMA((NPL,)),
            pltpu.SemaphoreType.DMA((NPL,)),
            pltpu.SemaphoreType.DMA((NPL,)),
            pltpu.SemaphoreType.DMA((NPL,)),
        ],
        compiler_params=pltpu.CompilerParams(collective_id=0),
    )(x)


# device time: 15738 ns/iter; 1.0945x vs baseline; 1.0945x over previous
import jax
import jax.numpy as jnp
from jax import lax
from jax.experimental import pallas as pl
from jax.experimental.pallas import tpu as pltpu

M = 1024
HALF = 512
ROWS = 512
CROWS = 128
NSELF = ROWS // CROWS
EXTRA = 1
NPULL = NSELF + EXTRA
NFWD = NSELF - EXTRA


def kernel(x):
    def body(
        x_ref,
        out_ref,
        in_all,
        send_y,
        recv_y,
        sum_buf,
        dsem,
        osem,
        ysend,
        yrecv,
        zsend,
        zrecv,
    ):
        my_x = lax.axis_index("x")
        my_y = lax.axis_index("y")
        my_z = lax.axis_index("z")
        ypeer = (my_x, 1 - my_y, my_z)
        znb = (my_x, my_y, 1 - my_z)

        row0 = my_z * ROWS
        other0 = (1 - my_z) * ROWS
        my_col = my_y * HALF
        peer_col = (1 - my_y) * HALF

        def chunk_row(c):
            if c < NSELF:
                return row0 + c * CROWS
            return other0 + (NSELF - EXTRA + (c - NSELF)) * CROWS

        barrier = pltpu.get_barrier_semaphore()
        for nbr in (ypeer, znb):
            pl.semaphore_signal(
                barrier, inc=1, device_id=nbr, device_id_type=pl.DeviceIdType.MESH
            )

        dmas = []
        for c in range(NPULL):
            rows = pl.ds(chunk_row(c), CROWS)
            dm = pltpu.make_async_copy(x_ref.at[0, rows, :], in_all.at[c], dsem.at[c])
            dm.start()
            dmas.append(dm)

        pl.semaphore_wait(barrier, 2)

        rdmas_y = []
        for c in range(NPULL):
            dmas[c].wait()
            send_y[c] = in_all[c, :, pl.ds(peer_col, HALF)].astype(jnp.bfloat16)
            ry = pltpu.make_async_remote_copy(
                src_ref=send_y.at[c],
                dst_ref=recv_y.at[c],
                send_sem=ysend.at[c],
                recv_sem=yrecv.at[c],
                device_id=ypeer,
                device_id_type=pl.DeviceIdType.MESH,
            )
            ry.start()
            rdmas_y.append(ry)

        rdmas_z = []
        copies = []
        for c in range(NPULL):
            rdmas_y[c].wait()
            rows = pl.ds(chunk_row(c), CROWS)
            sum_buf[c] = (
                in_all[c, :, pl.ds(my_col, HALF)].astype(jnp.bfloat16) + recv_y[c]
            )
            if c < NFWD:
                rz = pltpu.make_async_remote_copy(
                    src_ref=sum_buf.at[c],
                    dst_ref=out_ref.at[rows],
                    send_sem=zsend.at[c],
                    recv_sem=zrecv.at[c],
                    device_id=znb,
                    device_id_type=pl.DeviceIdType.MESH,
                )
                rz.start()
                rdmas_z.append(rz)
            cp = pltpu.make_async_copy(sum_buf.at[c], out_ref.at[rows], osem.at[c])
            cp.start()
            copies.append(cp)

        for cp in copies:
            cp.wait()
        for rz in rdmas_z:
            rz.wait()

    return pl.pallas_call(
        body,
        out_shape=jax.ShapeDtypeStruct((M, HALF), jnp.bfloat16),
        in_specs=[pl.BlockSpec(memory_space=pl.ANY)],
        out_specs=pl.BlockSpec(memory_space=pl.ANY),
        scratch_shapes=[
            pltpu.VMEM((NPULL, CROWS, 2 * HALF), jnp.float32),
            pltpu.VMEM((NPULL, CROWS, HALF), jnp.bfloat16),
            pltpu.VMEM((NPULL, CROWS, HALF), jnp.bfloat16),
            pltpu.VMEM((NPULL, CROWS, HALF), jnp.bfloat16),
            pltpu.SemaphoreType.DMA((NPULL,)),
            pltpu.SemaphoreType.DMA((NPULL,)),
            pltpu.SemaphoreType.DMA((NPULL,)),
            pltpu.SemaphoreType.DMA((NPULL,)),
            pltpu.SemaphoreType.DMA((NFWD,)),
            pltpu.SemaphoreType.DMA((NFWD,)),
        ],
        compiler_params=pltpu.CompilerParams(collective_id=0),
    )(x)
